# Initial kernel scaffold; baseline (speedup 1.0000x reference)
import jax
import jax.numpy as jnp
from jax.experimental import pallas as pl


def kernel(queries, neural_pts, neural_feats, WF0, bF0, WF1, bF1, WF2, bF2, WF3, bF3, WT, bT):
    raise NotImplementedError("write your pallas kernel here")


# speedup vs baseline: 1.8499x; 1.8499x over previous
import functools

import jax
import jax.numpy as jnp
from jax import lax
from jax.experimental import pallas as pl
from jax.experimental.pallas import tpu as pltpu
from jax.experimental.pallas import tpu_sc as plsc

KNN = 8
NPTS = 32768
NQ = 4096
FEAT = 32
HID = 256

QB = 128
SLAB = 2048
NSLAB = NPTS // SLAB
BIGF = 3.0e38


def _topk_body(q_ref, pts3_ref, idx_ref, d_ref):
    q = q_ref[...]
    qsq = jnp.sum(q * q, axis=1, keepdims=True)

    def fill(s, carry):
        p = pts3_ref[s]
        qp = lax.dot_general(q, p, (((1,), (0,)), ((), ())),
                             preferred_element_type=jnp.float32)
        psq = jnp.sum(p * p, axis=0, keepdims=True)
        d_ref[s] = qsq - 2.0 * qp + psq
        return carry

    lax.fori_loop(0, NSLAB, fill, 0)

    cols = []
    for _ in range(KNN):
        def minstep(s, m):
            return jnp.minimum(m, jnp.min(d_ref[s], axis=1, keepdims=True))
        m = lax.fori_loop(0, NSLAB, minstep, jnp.full((QB, 1), BIGF))

        def idxmask(s, imin):
            d = d_ref[s]
            hit = d == m
            col = lax.broadcasted_iota(jnp.int32, (QB, SLAB), 1) + s * SLAB
            here = jnp.min(jnp.where(hit, col, NPTS), axis=1, keepdims=True)
            d_ref[s] = jnp.where(hit, BIGF, d)
            return jnp.minimum(imin, here)

        i = lax.fori_loop(0, NSLAB, idxmask,
                          jnp.full((QB, 1), NPTS, dtype=jnp.int32))
        cols.append(i)
    idx_ref[...] = jnp.concatenate(cols, axis=1)


def _topk(queries, neural_pts):
    pts3 = neural_pts.T.reshape(3, NSLAB, SLAB).transpose(1, 0, 2)
    return pl.pallas_call(
        _topk_body,
        grid=(NQ // QB,),
        in_specs=[
            pl.BlockSpec((QB, 3), lambda i: (i, 0)),
            pl.BlockSpec((NSLAB, 3, SLAB), lambda i: (0, 0, 0)),
        ],
        out_specs=pl.BlockSpec((QB, KNN), lambda i: (i, 0)),
        out_shape=jax.ShapeDtypeStruct((NQ, KNN), jnp.int32),
        scratch_shapes=[pltpu.VMEM((NSLAB, QB, SLAB), jnp.float32)],
        compiler_params=pltpu.CompilerParams(
            dimension_semantics=("arbitrary",),
            vmem_limit_bytes=100 * 1024 * 1024,
        ),
    )(queries, pts3)


ROWD = 128
NIDX = NQ * KNN
GCH = 512


def _sc_gather(table, idx_flat):
    info = plsc.get_sparse_core_info()
    nw = info.num_cores * info.num_subcores
    bpw = NIDX // nw
    mesh = plsc.VectorSubcoreMesh(core_axis_name="c", subcore_axis_name="s")

    @functools.partial(
        pl.kernel, mesh=mesh,
        out_type=jax.ShapeDtypeStruct((NIDX, ROWD), jnp.float32),
        scratch_types=[
            pltpu.VMEM((GCH,), jnp.int32),
            pltpu.VMEM((GCH, ROWD), jnp.float32),
            pltpu.SemaphoreType.DMA,
        ],
    )
    def k(table_hbm, idx_hbm, out_hbm, idx_v, rows_v, sem):
        wid = lax.axis_index("s") * info.num_cores + lax.axis_index("c")
        for c in range(bpw // GCH):
            base = wid * bpw + c * GCH
            pltpu.sync_copy(idx_hbm.at[pl.ds(base, GCH)], idx_v)
            pltpu.async_copy(table_hbm.at[idx_v], rows_v, sem).wait()
            pltpu.sync_copy(rows_v, out_hbm.at[pl.ds(base, GCH)])

    return k(table, idx_flat)


QB2 = 512
RB = QB2 * KNN


def _lrelu(x):
    return jnp.where(x >= 0, x, 0.01 * x)


def _mlp_body(qrep_ref, rows_ref, w0_ref, b0_ref, w1_ref, b1_ref,
              w2_ref, b2_ref, w3_ref, b3_ref, wt_ref, bt_ref, out_ref):
    rows = rows_ref[...]
    pos = rows[:, 0:3]
    feat = rows[:, 3:3 + FEAT]
    x = qrep_ref[...] - pos
    pieces = [x]
    for i in range(4):
        f = 2.0 ** i
        pieces.append(jnp.sin(x * f))
        pieces.append(jnp.cos(x * f))
    pieces.append(feat)
    pieces.append(jnp.zeros((RB, 5), jnp.float32))
    h = jnp.concatenate(pieces, axis=1)
    h = _lrelu(jnp.dot(h, w0_ref[...], preferred_element_type=jnp.float32)
               + b0_ref[...])
    h = _lrelu(jnp.dot(h, w1_ref[...], preferred_element_type=jnp.float32)
               + b1_ref[...])
    h = _lrelu(jnp.dot(h, w2_ref[...], preferred_element_type=jnp.float32)
               + b2_ref[...])
    h = jnp.dot(h, w3_ref[...], preferred_element_type=jnp.float32) + b3_ref[...]
    sdf = jnp.dot(h, wt_ref[...], preferred_element_type=jnp.float32) + bt_ref[...]
    d2 = jnp.sum(x * x, axis=1, keepdims=True)
    dist = jnp.maximum(jnp.sqrt(d2), 1e-12)
    w = 1.0 / dist
    rowq = lax.broadcasted_iota(jnp.int32, (QB2, RB), 1) // KNN
    qid = lax.broadcasted_iota(jnp.int32, (QB2, RB), 0)
    g = jnp.where(rowq == qid, 1.0, 0.0)
    pair = jnp.concatenate([w * sdf, w], axis=1)
    agg = jnp.dot(g, pair, preferred_element_type=jnp.float32)
    out_ref[...] = agg[:, 0:1] / agg[:, 1:2]


def _mlp(qrep, rows, w0p, b0, w1, b1, w2, b2, w3, b3, wt, bt):
    grid = (NQ // QB2,)
    wspec = lambda shape: pl.BlockSpec(shape, lambda i: tuple(0 for _ in shape))
    return pl.pallas_call(
        _mlp_body,
        grid=grid,
        in_specs=[
            pl.BlockSpec((RB, 3), lambda i: (i, 0)),
            pl.BlockSpec((RB, ROWD), lambda i: (i, 0)),
            wspec((64, HID)), wspec((HID,)),
            wspec((HID, HID)), wspec((HID,)),
            wspec((HID, HID)), wspec((HID,)),
            wspec((HID, HID)), wspec((HID,)),
            wspec((HID, 1)), wspec((1,)),
        ],
        out_specs=pl.BlockSpec((QB2, 1), lambda i: (i, 0)),
        out_shape=jax.ShapeDtypeStruct((NQ, 1), jnp.float32),
        compiler_params=pltpu.CompilerParams(
            dimension_semantics=("arbitrary",),
            vmem_limit_bytes=100 * 1024 * 1024,
        ),
    )(qrep, rows, w0p, b0, w1, b1, w2, b2, w3, b3, wt, bt)


def kernel(queries, neural_pts, neural_feats, WF0, bF0, WF1, bF1,
           WF2, bF2, WF3, bF3, WT, bT):
    idx = _topk(queries, neural_pts)
    table = jnp.concatenate(
        [neural_pts, neural_feats.astype(jnp.float32),
         jnp.zeros((NPTS, ROWD - 3 - FEAT), jnp.float32)], axis=1)
    rows = _sc_gather(table, idx.reshape(NIDX))
    qrep = jnp.repeat(queries, KNN, axis=0)
    w0p = jnp.pad(WF0, ((0, 64 - WF0.shape[0]), (0, 0)))
    return _mlp(qrep, rows, w0p, bF0, WF1, bF1, WF2, bF2, WF3, bF3, WT, bT)


# speedup vs baseline: 5.2964x; 2.8631x over previous
import functools

import jax
import jax.numpy as jnp
from jax import lax
from jax.experimental import pallas as pl
from jax.experimental.pallas import tpu as pltpu
from jax.experimental.pallas import tpu_sc as plsc

KNN = 8
NPTS = 32768
NQ = 4096
FEAT = 32
HID = 256

QB = 128
CH = 128
NCHUNK = NPTS // CH
SLAB = 2048
NSLAB = NPTS // SLAB
CPS = SLAB // CH
BIGF = 3.0e38


def _chunk_body(qT_ref, pts_ref, cid_ref, m_ref):
    q2b = (qT_ref[...] * -2.0).astype(jnp.bfloat16)

    def slab_step(s, carry):
        p = pts_ref[pl.ds(s * SLAB, SLAB), :]
        qp2 = lax.dot_general(p.astype(jnp.bfloat16), q2b,
                              (((1,), (0,)), ((), ())),
                              preferred_element_type=jnp.float32)
        psq = jnp.sum(p * p, axis=1, keepdims=True)
        t = qp2 + psq
        m_ref[pl.ds(s * CPS, CPS), :] = jnp.min(
            t.reshape(CPS, CH, QB), axis=1)
        return carry

    lax.fori_loop(0, NSLAB, slab_step, 0)

    m = m_ref[...]
    rid = lax.broadcasted_iota(jnp.int32, (NCHUNK, QB), 0)
    cols = []
    for _ in range(KNN):
        mv = jnp.min(m, axis=0, keepdims=True)
        hit = m == mv
        cols.append(jnp.min(jnp.where(hit, rid, NCHUNK), axis=0,
                            keepdims=True))
        m = jnp.where(hit, BIGF, m)
    cid_ref[...] = jnp.concatenate(cols, axis=0)


def _chunk_topk(queries, neural_pts):
    return pl.pallas_call(
        _chunk_body,
        grid=(NQ // QB,),
        in_specs=[
            pl.BlockSpec((3, QB), lambda i: (0, i)),
            pl.BlockSpec((NPTS, 3), lambda i: (0, 0)),
        ],
        out_specs=pl.BlockSpec((KNN, QB), lambda i: (0, i)),
        out_shape=jax.ShapeDtypeStruct((KNN, NQ), jnp.int32),
        scratch_shapes=[pltpu.VMEM((NCHUNK, QB), jnp.float32)],
        compiler_params=pltpu.CompilerParams(
            dimension_semantics=("arbitrary",),
            vmem_limit_bytes=100 * 1024 * 1024,
        ),
    )(queries.T, neural_pts)


CAND_D = 4 * CH
QB3 = 128


def _refine_body(q_ref, cand_ref, idx_ref):
    qx = q_ref[:, 0:1]
    qy = q_ref[:, 1:2]
    qz = q_ref[:, 2:3]
    qsq = (qx * qx + qy * qy) + qz * qz
    qbx = qx.astype(jnp.bfloat16).astype(jnp.float32)
    qby = qy.astype(jnp.bfloat16).astype(jnp.float32)
    qbz = qz.astype(jnp.bfloat16).astype(jnp.float32)
    d2s, gs = [], []
    for j in range(KNN):
        base = j * CAND_D
        x = cand_ref[:, base:base + CH]
        y = cand_ref[:, base + CH:base + 2 * CH]
        z = cand_ref[:, base + 2 * CH:base + 3 * CH]
        gs.append(cand_ref[:, base + 3 * CH:base + 4 * CH])
        qp = (qbx * x.astype(jnp.bfloat16).astype(jnp.float32)
              + qby * y.astype(jnp.bfloat16).astype(jnp.float32)) \
            + qbz * z.astype(jnp.bfloat16).astype(jnp.float32)
        psq = (x * x + y * y) + z * z
        d2s.append((qsq - 2.0 * qp) + psq)
    d2 = jnp.concatenate(d2s, axis=1)
    gid = jnp.concatenate(gs, axis=1).astype(jnp.int32)
    cols = []
    for _ in range(KNN):
        mv = jnp.min(d2, axis=1, keepdims=True)
        hit = d2 == mv
        cols.append(jnp.min(jnp.where(hit, gid, NPTS), axis=1,
                            keepdims=True))
        d2 = jnp.where(hit, BIGF, d2)
    idx_ref[...] = jnp.concatenate(cols, axis=1)


def _refine_topk(queries, cand_merged):
    return pl.pallas_call(
        _refine_body,
        grid=(NQ // QB3,),
        in_specs=[
            pl.BlockSpec((QB3, 3), lambda i: (i, 0)),
            pl.BlockSpec((QB3, KNN * CAND_D), lambda i: (i, 0)),
        ],
        out_specs=pl.BlockSpec((QB3, KNN), lambda i: (i, 0)),
        out_shape=jax.ShapeDtypeStruct((NQ, KNN), jnp.int32),
        compiler_params=pltpu.CompilerParams(
            dimension_semantics=("arbitrary",),
            vmem_limit_bytes=100 * 1024 * 1024,
        ),
    )(queries, cand_merged)


ROWD = 128
NIDX = NQ * KNN
GCH = 512


def _sc_gather(table, idx_flat, rowd, gch):
    info = plsc.get_sparse_core_info()
    nw = info.num_cores * info.num_subcores
    bpw = NIDX // nw
    mesh = plsc.VectorSubcoreMesh(core_axis_name="c", subcore_axis_name="s")

    @functools.partial(
        pl.kernel, mesh=mesh,
        out_type=jax.ShapeDtypeStruct((NIDX, rowd), jnp.float32),
        scratch_types=[
            pltpu.VMEM((gch,), jnp.int32),
            pltpu.VMEM((gch, rowd), jnp.float32),
            pltpu.SemaphoreType.DMA,
        ],
    )
    def k(table_hbm, idx_hbm, out_hbm, idx_v, rows_v, sem):
        wid = lax.axis_index("s") * info.num_cores + lax.axis_index("c")
        for c in range(bpw // gch):
            base = wid * bpw + c * gch
            pltpu.sync_copy(idx_hbm.at[pl.ds(base, gch)], idx_v)
            pltpu.async_copy(table_hbm.at[idx_v], rows_v, sem).wait()
            pltpu.sync_copy(rows_v, out_hbm.at[pl.ds(base, gch)])

    return k(table, idx_flat)


QB2 = 512
RB = QB2 * KNN


def _lrelu(x):
    return jnp.where(x >= 0, x, 0.01 * x)


def _mlp_body(qrep_ref, rows_ref, w0_ref, b0_ref, w1_ref, b1_ref,
              w2_ref, b2_ref, w3_ref, b3_ref, wt_ref, bt_ref, out_ref):
    rows = rows_ref[...]
    pos = rows[:, 0:3]
    feat = rows[:, 3:3 + FEAT]
    x = qrep_ref[...] - pos
    pieces = [x]
    for i in range(4):
        f = 2.0 ** i
        pieces.append(jnp.sin(x * f))
        pieces.append(jnp.cos(x * f))
    pieces.append(feat)
    pieces.append(jnp.zeros((RB, 5), jnp.float32))
    h = jnp.concatenate(pieces, axis=1)
    h = _lrelu(jnp.dot(h, w0_ref[...], preferred_element_type=jnp.float32)
               + b0_ref[...])
    h = _lrelu(jnp.dot(h, w1_ref[...], preferred_element_type=jnp.float32)
               + b1_ref[...])
    h = _lrelu(jnp.dot(h, w2_ref[...], preferred_element_type=jnp.float32)
               + b2_ref[...])
    h = jnp.dot(h, w3_ref[...], preferred_element_type=jnp.float32) + b3_ref[...]
    sdf = jnp.dot(h, wt_ref[...], preferred_element_type=jnp.float32) + bt_ref[...]
    d2 = jnp.sum(x * x, axis=1, keepdims=True)
    dist = jnp.maximum(jnp.sqrt(d2), 1e-12)
    w = 1.0 / dist
    rowq = lax.broadcasted_iota(jnp.int32, (QB2, RB), 1) // KNN
    qid = lax.broadcasted_iota(jnp.int32, (QB2, RB), 0)
    g = jnp.where(rowq == qid, 1.0, 0.0)
    pair = jnp.concatenate([w * sdf, w], axis=1)
    agg = jnp.dot(g, pair, preferred_element_type=jnp.float32)
    out_ref[...] = agg[:, 0:1] / agg[:, 1:2]


def _mlp(qrep, rows, w0p, b0, w1, b1, w2, b2, w3, b3, wt, bt):
    grid = (NQ // QB2,)
    wspec = lambda shape: pl.BlockSpec(shape, lambda i: tuple(0 for _ in shape))
    return pl.pallas_call(
        _mlp_body,
        grid=grid,
        in_specs=[
            pl.BlockSpec((RB, 3), lambda i: (i, 0)),
            pl.BlockSpec((RB, ROWD), lambda i: (i, 0)),
            wspec((64, HID)), wspec((HID,)),
            wspec((HID, HID)), wspec((HID,)),
            wspec((HID, HID)), wspec((HID,)),
            wspec((HID, HID)), wspec((HID,)),
            wspec((HID, 1)), wspec((1,)),
        ],
        out_specs=pl.BlockSpec((QB2, 1), lambda i: (i, 0)),
        out_shape=jax.ShapeDtypeStruct((NQ, 1), jnp.float32),
        compiler_params=pltpu.CompilerParams(
            dimension_semantics=("arbitrary",),
            vmem_limit_bytes=100 * 1024 * 1024,
        ),
    )(qrep, rows, w0p, b0, w1, b1, w2, b2, w3, b3, wt, bt)


def kernel(queries, neural_pts, neural_feats, WF0, bF0, WF1, bF1,
           WF2, bF2, WF3, bF3, WT, bT):
    qrep = jnp.repeat(queries, KNN, axis=0)
    cids = _chunk_topk(queries, neural_pts).T.reshape(NIDX)
    chunk_tab = jnp.concatenate(
        [neural_pts.reshape(NCHUNK, CH, 3).transpose(0, 2, 1).reshape(
            NCHUNK, 3 * CH),
         jnp.arange(NPTS, dtype=jnp.float32).reshape(NCHUNK, CH)], axis=1)
    cand = _sc_gather(chunk_tab, cids, CAND_D, 128)
    idx = _refine_topk(queries, cand.reshape(NQ, KNN * CAND_D))
    table = jnp.concatenate(
        [neural_pts, neural_feats.astype(jnp.float32),
         jnp.zeros((NPTS, ROWD - 3 - FEAT), jnp.float32)], axis=1)
    rows = _sc_gather(table, idx.reshape(NIDX), ROWD, GCH)
    w0p = jnp.pad(WF0, ((0, 64 - WF0.shape[0]), (0, 0)))
    return _mlp(qrep, rows, w0p, bF0, WF1, bF1, WF2, bF2, WF3, bF3, WT, bT)


# speedup vs baseline: 7.8744x; 1.4867x over previous
import functools

import jax
import jax.numpy as jnp
from jax import lax
from jax.experimental import pallas as pl
from jax.experimental.pallas import tpu as pltpu
from jax.experimental.pallas import tpu_sc as plsc

KNN = 8
NPTS = 32768
NQ = 4096
FEAT = 32
HID = 256

QB = 128
CH = 128
NCHUNK = NPTS // CH
SLAB = 2048
NSLAB = NPTS // SLAB
CPS = SLAB // CH
BIGF = 3.0e38


def _chunk_body(qT_ref, pts_ref, cid_ref, m_ref, psq_ref):
    q2b = (qT_ref[...] * -2.0).astype(jnp.bfloat16)

    @pl.when(pl.program_id(0) == 0)
    def _fill_psq():
        def psq_step(s, carry):
            p = pts_ref[pl.ds(s * SLAB, SLAB), :]
            psq_ref[pl.ds(s * SLAB, SLAB), :] = jnp.sum(
                p * p, axis=1, keepdims=True)
            return carry
        lax.fori_loop(0, NSLAB, psq_step, 0)

    def slab_step(s, carry):
        p = pts_ref[pl.ds(s * SLAB, SLAB), :]
        qp2 = lax.dot_general(p.astype(jnp.bfloat16), q2b,
                              (((1,), (0,)), ((), ())),
                              preferred_element_type=jnp.float32)
        t = qp2 + psq_ref[pl.ds(s * SLAB, SLAB), :]
        m_ref[pl.ds(s * CPS, CPS), :] = jnp.min(
            t.reshape(CPS, CH, QB), axis=1)
        return carry

    lax.fori_loop(0, NSLAB, slab_step, 0)

    m = m_ref[...]
    rid = lax.broadcasted_iota(jnp.int32, (NCHUNK, QB), 0)
    cols = []
    for _ in range(KNN):
        mv = jnp.min(m, axis=0, keepdims=True)
        hit = m == mv
        cols.append(jnp.min(jnp.where(hit, rid, NCHUNK), axis=0,
                            keepdims=True))
        m = jnp.where(hit, BIGF, m)
    cid_ref[...] = jnp.concatenate(cols, axis=0)


def _chunk_topk(queries, neural_pts):
    return pl.pallas_call(
        _chunk_body,
        grid=(NQ // QB,),
        in_specs=[
            pl.BlockSpec((3, QB), lambda i: (0, i)),
            pl.BlockSpec((NPTS, 3), lambda i: (0, 0)),
        ],
        out_specs=pl.BlockSpec((KNN, QB), lambda i: (0, i)),
        out_shape=jax.ShapeDtypeStruct((KNN, NQ), jnp.int32),
        scratch_shapes=[pltpu.VMEM((NCHUNK, QB), jnp.float32),
                        pltpu.VMEM((NPTS, 1), jnp.float32)],
        compiler_params=pltpu.CompilerParams(
            dimension_semantics=("arbitrary",),
            vmem_limit_bytes=100 * 1024 * 1024,
        ),
    )(queries.T, neural_pts)


CAND_D = 3 * CH
QB3 = 128


def _refine_body(q_ref, cids_ref, cand_ref, idx_ref):
    qx = q_ref[:, 0:1]
    qy = q_ref[:, 1:2]
    qz = q_ref[:, 2:3]
    qsq = (qx * qx + qy * qy) + qz * qz
    qbx = qx.astype(jnp.bfloat16).astype(jnp.float32)
    qby = qy.astype(jnp.bfloat16).astype(jnp.float32)
    qbz = qz.astype(jnp.bfloat16).astype(jnp.float32)
    lane = lax.broadcasted_iota(jnp.int32, (QB3, CH), 1)
    d2s, gs = [], []
    for j in range(KNN):
        base = j * CAND_D
        x = cand_ref[:, base:base + CH]
        y = cand_ref[:, base + CH:base + 2 * CH]
        z = cand_ref[:, base + 2 * CH:base + 3 * CH]
        gs.append(cids_ref[:, j:j + 1] * CH + lane)
        qp = (qbx * x.astype(jnp.bfloat16).astype(jnp.float32)
              + qby * y.astype(jnp.bfloat16).astype(jnp.float32)) \
            + qbz * z.astype(jnp.bfloat16).astype(jnp.float32)
        psq = (x * x + y * y) + z * z
        d2s.append((qsq - 2.0 * qp) + psq)
    d2 = jnp.concatenate(d2s, axis=1)
    gid = jnp.concatenate(gs, axis=1)
    cols = []
    for _ in range(KNN):
        mv = jnp.min(d2, axis=1, keepdims=True)
        hit = d2 == mv
        cols.append(jnp.min(jnp.where(hit, gid, NPTS), axis=1,
                            keepdims=True))
        d2 = jnp.where(hit, BIGF, d2)
    idx_ref[...] = jnp.concatenate(cols, axis=1)


def _refine_topk(queries, cids2d, cand_merged):
    return pl.pallas_call(
        _refine_body,
        grid=(NQ // QB3,),
        in_specs=[
            pl.BlockSpec((QB3, 3), lambda i: (i, 0)),
            pl.BlockSpec((QB3, KNN), lambda i: (i, 0)),
            pl.BlockSpec((QB3, KNN * CAND_D), lambda i: (i, 0)),
        ],
        out_specs=pl.BlockSpec((QB3, KNN), lambda i: (i, 0)),
        out_shape=jax.ShapeDtypeStruct((NQ, KNN), jnp.int32),
        compiler_params=pltpu.CompilerParams(
            dimension_semantics=("arbitrary",),
            vmem_limit_bytes=100 * 1024 * 1024,
        ),
    )(queries, cids2d, cand_merged)


ROWD = 128
NIDX = NQ * KNN
GCH = 512


def _sc_gather(table, idx_flat, rowd, gch):
    info = plsc.get_sparse_core_info()
    nw = info.num_cores * info.num_subcores
    bpw = NIDX // nw
    mesh = plsc.VectorSubcoreMesh(core_axis_name="c", subcore_axis_name="s")

    @functools.partial(
        pl.kernel, mesh=mesh,
        out_type=jax.ShapeDtypeStruct((NIDX, rowd), jnp.float32),
        scratch_types=[
            pltpu.VMEM((gch,), jnp.int32),
            pltpu.VMEM((gch, rowd), jnp.float32),
            pltpu.SemaphoreType.DMA,
        ],
    )
    def k(table_hbm, idx_hbm, out_hbm, idx_v, rows_v, sem):
        wid = lax.axis_index("s") * info.num_cores + lax.axis_index("c")
        for c in range(bpw // gch):
            base = wid * bpw + c * gch
            pltpu.sync_copy(idx_hbm.at[pl.ds(base, gch)], idx_v)
            pltpu.async_copy(table_hbm.at[idx_v], rows_v, sem).wait()
            pltpu.sync_copy(rows_v, out_hbm.at[pl.ds(base, gch)])

    return k(table, idx_flat)


QB2 = 512
RB = QB2 * KNN


def _lrelu(x):
    return jnp.where(x >= 0, x, 0.01 * x)


def _mlp_body(qrep_ref, rows_ref, w0_ref, b0_ref, w1_ref, b1_ref,
              w2_ref, b2_ref, w3_ref, b3_ref, wt_ref, bt_ref, out_ref):
    rows = rows_ref[...]
    pos = rows[:, 0:3]
    feat = rows[:, 3:3 + FEAT]
    x = qrep_ref[...] - pos
    s1 = jnp.sin(x)
    c1 = jnp.cos(x)
    s2 = 2.0 * s1 * c1
    c2 = 1.0 - 2.0 * s1 * s1
    s4 = 2.0 * s2 * c2
    c4 = 1.0 - 2.0 * s2 * s2
    s8 = 2.0 * s4 * c4
    c8 = 1.0 - 2.0 * s4 * s4
    pieces = [x, s1, c1, s2, c2, s4, c4, s8, c8, feat]
    pieces.append(jnp.zeros((RB, 5), jnp.float32))
    h = jnp.concatenate(pieces, axis=1)
    h = _lrelu(jnp.dot(h, w0_ref[...], preferred_element_type=jnp.float32)
               + b0_ref[...])
    h = _lrelu(jnp.dot(h, w1_ref[...], preferred_element_type=jnp.float32)
               + b1_ref[...])
    h = _lrelu(jnp.dot(h, w2_ref[...], preferred_element_type=jnp.float32)
               + b2_ref[...])
    h = jnp.dot(h, w3_ref[...], preferred_element_type=jnp.float32) + b3_ref[...]
    sdf = jnp.dot(h, wt_ref[...], preferred_element_type=jnp.float32) + bt_ref[...]
    d2 = jnp.sum(x * x, axis=1, keepdims=True)
    dist = jnp.maximum(jnp.sqrt(d2), 1e-12)
    w = 1.0 / dist
    rowq = lax.broadcasted_iota(jnp.int32, (QB2, RB), 1) // KNN
    qid = lax.broadcasted_iota(jnp.int32, (QB2, RB), 0)
    g = jnp.where(rowq == qid, 1.0, 0.0)
    pair = jnp.concatenate([w * sdf, w], axis=1)
    agg = jnp.dot(g, pair, preferred_element_type=jnp.float32)
    out_ref[...] = agg[:, 0:1] / agg[:, 1:2]


def _mlp(qrep, rows, w0p, b0, w1, b1, w2, b2, w3, b3, wt, bt):
    grid = (NQ // QB2,)
    wspec = lambda shape: pl.BlockSpec(shape, lambda i: tuple(0 for _ in shape))
    return pl.pallas_call(
        _mlp_body,
        grid=grid,
        in_specs=[
            pl.BlockSpec((RB, 3), lambda i: (i, 0)),
            pl.BlockSpec((RB, ROWD), lambda i: (i, 0)),
            wspec((64, HID)), wspec((HID,)),
            wspec((HID, HID)), wspec((HID,)),
            wspec((HID, HID)), wspec((HID,)),
            wspec((HID, HID)), wspec((HID,)),
            wspec((HID, 1)), wspec((1,)),
        ],
        out_specs=pl.BlockSpec((QB2, 1), lambda i: (i, 0)),
        out_shape=jax.ShapeDtypeStruct((NQ, 1), jnp.float32),
        compiler_params=pltpu.CompilerParams(
            dimension_semantics=("arbitrary",),
            vmem_limit_bytes=100 * 1024 * 1024,
        ),
    )(qrep, rows, w0p, b0, w1, b1, w2, b2, w3, b3, wt, bt)


def kernel(queries, neural_pts, neural_feats, WF0, bF0, WF1, bF1,
           WF2, bF2, WF3, bF3, WT, bT):
    qrep = jnp.repeat(queries, KNN, axis=0)
    cids2d = _chunk_topk(queries, neural_pts).T
    chunk_tab = neural_pts.reshape(NCHUNK, CH, 3).transpose(0, 2, 1).reshape(
        NCHUNK, 3 * CH)
    cand = _sc_gather(chunk_tab, cids2d.reshape(NIDX), CAND_D, 128)
    idx = _refine_topk(queries, cids2d, cand.reshape(NQ, KNN * CAND_D))
    table = jnp.concatenate(
        [neural_pts, neural_feats.astype(jnp.float32),
         jnp.zeros((NPTS, ROWD - 3 - FEAT), jnp.float32)], axis=1)
    rows = _sc_gather(table, idx.reshape(NIDX), ROWD, GCH)
    w0p = jnp.pad(WF0, ((0, 64 - WF0.shape[0]), (0, 0)))
    return _mlp(qrep, rows, w0p, bF0, WF1, bF1, WF2, bF2, WF3, bF3, WT, bT)
